# baseline (speedup 1.0000x reference)
import functools

import jax
import jax.numpy as jnp
from jax import lax
from jax.experimental import pallas as pl
from jax.experimental.pallas import tpu as pltpu
from jax.experimental.pallas import tpu_sc as plsc

_B, _S, _D = 4, 4096, 1024
_NC, _NS = 2, 16
_NW = _NC * _NS
_ROWS = _S // _NW
_SIZES = (8, 48, 40, 32)
_SRC = (0, 8, 56, 96)
_DST = (0, 8, 56, 8)

_mesh = plsc.VectorSubcoreMesh(core_axis_name="c", subcore_axis_name="s")


@functools.partial(
    pl.kernel,
    mesh=_mesh,
    out_type=jax.ShapeDtypeStruct((_B * _S, _D), jnp.float32),
    scratch_types=[
        pltpu.VMEM((96, _D), jnp.float32),
        pltpu.SemaphoreType.DMA((len(_SIZES),)),
        pltpu.SemaphoreType.DMA((len(_SIZES),)),
    ],
)
def _bcast_rows(table_hbm, out_hbm, buf, in_sems, out_sems):
    wid = lax.axis_index("s") * _NC + lax.axis_index("c")
    base = wid * _ROWS

    def start_read(i):
        return pltpu.async_copy(
            table_hbm.at[pl.ds(base + _SRC[i], _SIZES[i])],
            buf.at[pl.ds(_DST[i], _SIZES[i])],
            in_sems.at[i],
        )

    def fire_writes(i):
        return [
            pltpu.async_copy(
                buf.at[pl.ds(_DST[i], _SIZES[i])],
                out_hbm.at[pl.ds(b * _S + base + _SRC[i], _SIZES[i])],
                out_sems.at[i],
            )
            for b in range(_B)
        ]

    reads = [start_read(0), start_read(1), start_read(2)]
    reads[0].wait()
    writes0 = fire_writes(0)
    reads[1].wait()
    writes1 = fire_writes(1)
    reads[2].wait()
    writes2 = fire_writes(2)
    for cp in writes1:
        cp.wait()
    start_read(3).wait()
    writes3 = fire_writes(3)
    for cp in writes0 + writes2 + writes3:
        cp.wait()


def kernel(x, table):
    del x
    out = _bcast_rows(table)
    return out.reshape(_B, _S, _D)


# speedup vs baseline: 1.0121x; 1.0121x over previous
import functools

import jax
import jax.numpy as jnp
from jax import lax
from jax.experimental import pallas as pl
from jax.experimental.pallas import tpu as pltpu
from jax.experimental.pallas import tpu_sc as plsc

_B, _S, _D = 4, 4096, 1024
_NC, _NS = 2, 16
_NW = _NC * _NS
_ROWS = _S // _NW
_SIZES = (48, 40, 40)
_SRC = (0, 48, 88)
_DST = (0, 48, 0)

_mesh = plsc.VectorSubcoreMesh(core_axis_name="c", subcore_axis_name="s")


@functools.partial(
    pl.kernel,
    mesh=_mesh,
    out_type=jax.ShapeDtypeStruct((_B * _S, _D), jnp.float32),
    scratch_types=[
        pltpu.VMEM((88, _D), jnp.float32),
        pltpu.SemaphoreType.DMA((len(_SIZES),)),
        pltpu.SemaphoreType.DMA((len(_SIZES),)),
    ],
)
def _bcast_rows(table_hbm, out_hbm, buf, in_sems, out_sems):
    wid = lax.axis_index("s") * _NC + lax.axis_index("c")
    base = wid * _ROWS

    def start_read(i):
        return pltpu.async_copy(
            table_hbm.at[pl.ds(base + _SRC[i], _SIZES[i])],
            buf.at[pl.ds(_DST[i], _SIZES[i])],
            in_sems.at[i],
        )

    def fire_writes(i):
        return [
            pltpu.async_copy(
                buf.at[pl.ds(_DST[i], _SIZES[i])],
                out_hbm.at[pl.ds(b * _S + base + _SRC[i], _SIZES[i])],
                out_sems.at[i],
            )
            for b in range(_B)
        ]

    reads = [start_read(0), start_read(1)]
    reads[0].wait()
    writes0 = fire_writes(0)
    reads[1].wait()
    writes1 = fire_writes(1)
    for cp in writes0:
        cp.wait()
    start_read(2).wait()
    writes2 = fire_writes(2)
    for cp in writes1 + writes2:
        cp.wait()


def kernel(x, table):
    del x
    out = _bcast_rows(table)
    return out.reshape(_B, _S, _D)
